# baseline (speedup 1.0000x reference)
import functools

import jax
import jax.numpy as jnp
from jax import lax
from jax.experimental import pallas as pl

B, L, D = 256, 128, 6
F_ATOM, F_BOND, FP = 39, 10, 128


def _lrelu(x):
    return jnp.where(x >= 0, x, 0.01 * x)


def _fused_kernel(atom_ref, bond_ref, aidx_ref, bidx_ref,
                  w_atom_ref, b_atom_ref, wnb_a_ref, wnb_b_ref, b_nb_ref,
                  w1_ref, w2_ref, b_align_ref, w_att_ref, b_att_ref,
                  w_ih_ref, b_ih_ref, w_hh_ref, b_hh_ref,
                  out_ref):
    atoms = atom_ref[0]
    bonds = bond_ref[0]
    aidx = aidx_ref[0]
    bidx = bidx_ref[0]

    dot = functools.partial(jnp.dot, preferred_element_type=jnp.float32)

    af = _lrelu(dot(atoms, w_atom_ref[...]) + b_atom_ref[...])

    align1 = jnp.sum(af * w1_ref[...], axis=1, keepdims=True) + b_align_ref[0, 0]

    w2 = w2_ref[...]
    b_nb = b_nb_ref[...]

    tp = jnp.concatenate([dot(atoms, wnb_a_ref[...]),
                          dot(bonds, wnb_b_ref[...])], axis=0)
    iota2 = lax.broadcasted_iota(jnp.int32, (L, 2 * L), 1)

    nfs = []
    scores = []
    valids = []
    for d in range(D):
        a_col = aidx[:, d:d + 1]
        b_col = bidx[:, d:d + 1]
        g = ((a_col == iota2) | (b_col + L == iota2)).astype(jnp.float32)
        nf = _lrelu(dot(g, tp) + b_nb)
        score = _lrelu(align1 + jnp.sum(nf * w2, axis=1, keepdims=True))
        valid = (a_col != L - 1)
        score = jnp.where(valid, score, score - 9e8)
        nfs.append(nf)
        scores.append(score)
        valids.append(valid.astype(jnp.float32))

    m = scores[0]
    for d in range(1, D):
        m = jnp.maximum(m, scores[d])
    exps = [jnp.exp(s - m) for s in scores]
    denom = exps[0]
    for d in range(1, D):
        denom = denom + exps[d]
    inv = 1.0 / denom

    acc = None
    tot = None
    for d in range(D):
        attn = exps[d] * inv * valids[d]
        term = attn * nfs[d]
        acc = term if acc is None else acc + term
        tot = attn if tot is None else tot + attn

    ctx_pre = dot(acc, w_att_ref[...]) + tot * b_att_ref[...]
    context = jnp.where(ctx_pre > 0, ctx_pre, jnp.exp(ctx_pre) - 1.0)

    gi = dot(context, w_ih_ref[...]) + b_ih_ref[...]
    gh = dot(af, w_hh_ref[...]) + b_hh_ref[...]
    r = jax.nn.sigmoid(gi[:, :FP] + gh[:, :FP])
    z = jax.nn.sigmoid(gi[:, FP:2 * FP] + gh[:, FP:2 * FP])
    n = jnp.tanh(gi[:, 2 * FP:] + r * gh[:, 2 * FP:])
    hnew = (1.0 - z) * n + z * af
    out_ref[0] = jnp.maximum(hnew, 0.0)


def kernel(atom_list, bond_list, atom_degree_list, bond_degree_list, atom_mask,
           W_atom, b_atom, W_nb, b_nb, W_align, b_align, W_att, b_att,
           W_ih, W_hh, b_ih, b_hh):
    del atom_mask
    aidx = atom_degree_list.astype(jnp.int32)
    bidx = bond_degree_list.astype(jnp.int32)

    wnb_a = W_nb[:F_ATOM]
    wnb_b = W_nb[F_ATOM:]
    w1 = W_align[:FP, 0].reshape(1, FP)
    w2 = W_align[FP:, 0].reshape(1, FP)
    b_align2 = b_align.reshape(1, 1)
    w_ih_t = W_ih.T
    w_hh_t = W_hh.T

    rep2 = lambda arr: pl.BlockSpec(arr.shape, lambda i: (0,) * arr.ndim)
    row = lambda v: v.reshape(1, -1)

    grid = (B,)
    out = pl.pallas_call(
        _fused_kernel,
        grid=grid,
        in_specs=[
            pl.BlockSpec((1, L, F_ATOM), lambda i: (i, 0, 0)),
            pl.BlockSpec((1, L, F_BOND), lambda i: (i, 0, 0)),
            pl.BlockSpec((1, L, D), lambda i: (i, 0, 0)),
            pl.BlockSpec((1, L, D), lambda i: (i, 0, 0)),
            rep2(W_atom), rep2(row(b_atom)),
            rep2(wnb_a), rep2(wnb_b), rep2(row(b_nb)),
            rep2(w1), rep2(w2), rep2(b_align2),
            rep2(W_att), rep2(row(b_att)),
            rep2(w_ih_t), rep2(row(b_ih)),
            rep2(w_hh_t), rep2(row(b_hh)),
        ],
        out_specs=pl.BlockSpec((1, L, FP), lambda i: (i, 0, 0)),
        out_shape=jax.ShapeDtypeStruct((B, L, FP), jnp.float32),
    )(atom_list, bond_list, aidx, bidx,
      W_atom, row(b_atom), wnb_a, wnb_b, row(b_nb),
      w1, w2, b_align2, W_att, row(b_att),
      w_ih_t, row(b_ih), w_hh_t, row(b_hh))
    return out


# speedup vs baseline: 1.0641x; 1.0641x over previous
import functools

import jax
import jax.numpy as jnp
from jax import lax
from jax.experimental import pallas as pl

B, L, D = 256, 128, 6
F_ATOM, F_BOND, FP = 39, 10, 128


def _lrelu(x):
    return jnp.where(x >= 0, x, 0.01 * x)


def _dgT(a, b):
    return lax.dot_general(a, b, (((0,), (0,)), ((), ())),
                           preferred_element_type=jnp.float32)


def _fused_kernel(atomT_ref, bondT_ref, aidxT_ref, bidxT_ref,
                  w_atom2_ref, wnb_a2_ref, wnb_b2_ref,
                  w1_ref, w2_ref, b_align_ref, w_att_ref, b_att_ref,
                  w_ihT_ref, b_ih_ref, w_hhT_ref, b_hh_ref,
                  out_ref):
    atomsT = atomT_ref[0]
    bondsT = bondT_ref[0]
    aidx = aidxT_ref[0]
    bidx = bidxT_ref[0]

    dot = functools.partial(jnp.dot, preferred_element_type=jnp.float32)

    afT = _lrelu(_dgT(w_atom2_ref[...], atomsT))
    align1 = _dgT(w1_ref[...], afT) + b_align_ref[0, 0]

    apT = _dgT(wnb_a2_ref[...], atomsT)
    bpT = _dgT(wnb_b2_ref[...], bondsT)
    tpT = jnp.concatenate([apT, bpT], axis=1)

    iota_s = lax.broadcasted_iota(jnp.int32, (2 * L, L), 0)
    w2 = w2_ref[...]

    nfs = []
    scores = []
    valids = []
    for d in range(D):
        arow = aidx[d:d + 1, :]
        brow = bidx[d:d + 1, :]
        g = ((arow == iota_s) | (brow + L == iota_s)).astype(jnp.float32)
        nfT = _lrelu(dot(tpT, g))
        score = _lrelu(align1 + _dgT(w2, nfT))
        valid = (arow != L - 1)
        score = jnp.where(valid, score, score - 9e8)
        nfs.append(nfT)
        scores.append(score)
        valids.append(valid.astype(jnp.float32))

    m = scores[0]
    for d in range(1, D):
        m = jnp.maximum(m, scores[d])
    exps = [jnp.exp(s - m) for s in scores]
    denom = exps[0]
    for d in range(1, D):
        denom = denom + exps[d]
    inv = 1.0 / denom

    acc = None
    tot = None
    for d in range(D):
        attn = exps[d] * inv * valids[d]
        term = attn * nfs[d]
        acc = term if acc is None else acc + term
        tot = attn if tot is None else tot + attn

    ctx_pre = _dgT(w_att_ref[...], acc) + b_att_ref[...] * tot
    ctxT = jnp.where(ctx_pre > 0, ctx_pre, jnp.exp(ctx_pre) - 1.0)

    gi = _dgT(ctxT, w_ihT_ref[...]) + b_ih_ref[...]
    gh = _dgT(afT, w_hhT_ref[...]) + b_hh_ref[...]
    r = jax.nn.sigmoid(gi[:, :FP] + gh[:, :FP])
    z = jax.nn.sigmoid(gi[:, FP:2 * FP] + gh[:, FP:2 * FP])
    n = jnp.tanh(gi[:, 2 * FP:] + r * gh[:, 2 * FP:])
    af = afT.T
    hnew = (1.0 - z) * n + z * af
    out_ref[0] = jnp.maximum(hnew, 0.0)


def kernel(atom_list, bond_list, atom_degree_list, bond_degree_list, atom_mask,
           W_atom, b_atom, W_nb, b_nb, W_align, b_align, W_att, b_att,
           W_ih, W_hh, b_ih, b_hh):
    del atom_mask
    ones = jnp.ones((B, 1, L), jnp.float32)
    atomsT2 = jnp.concatenate([atom_list.transpose(0, 2, 1), ones], axis=1)
    bondsT2 = jnp.concatenate([bond_list.transpose(0, 2, 1), ones], axis=1)
    aidxT = atom_degree_list.astype(jnp.int32).transpose(0, 2, 1)
    bidxT = bond_degree_list.astype(jnp.int32).transpose(0, 2, 1)

    w_atom2 = jnp.concatenate([W_atom, b_atom[None, :]], axis=0)
    half_bnb = 0.5 * b_nb[None, :]
    wnb_a2 = jnp.concatenate([W_nb[:F_ATOM], half_bnb], axis=0)
    wnb_b2 = jnp.concatenate([W_nb[F_ATOM:], half_bnb], axis=0)
    w1 = W_align[:FP]
    w2 = W_align[FP:]
    b_align2 = b_align.reshape(1, 1)
    b_att_col = b_att.reshape(FP, 1)
    w_ihT = W_ih.T
    w_hhT = W_hh.T

    rep = lambda arr: pl.BlockSpec(arr.shape, lambda i: (0,) * arr.ndim)
    row = lambda v: v.reshape(1, -1)

    out = pl.pallas_call(
        _fused_kernel,
        grid=(B,),
        in_specs=[
            pl.BlockSpec((1, F_ATOM + 1, L), lambda i: (i, 0, 0)),
            pl.BlockSpec((1, F_BOND + 1, L), lambda i: (i, 0, 0)),
            pl.BlockSpec((1, D, L), lambda i: (i, 0, 0)),
            pl.BlockSpec((1, D, L), lambda i: (i, 0, 0)),
            rep(w_atom2), rep(wnb_a2), rep(wnb_b2),
            rep(w1), rep(w2), rep(b_align2),
            rep(W_att), rep(b_att_col),
            rep(w_ihT), rep(row(b_ih)),
            rep(w_hhT), rep(row(b_hh)),
        ],
        out_specs=pl.BlockSpec((1, L, FP), lambda i: (i, 0, 0)),
        out_shape=jax.ShapeDtypeStruct((B, L, FP), jnp.float32),
    )(atomsT2, bondsT2, aidxT, bidxT,
      w_atom2, wnb_a2, wnb_b2, w1, w2, b_align2, W_att, b_att_col,
      w_ihT, row(b_ih), w_hhT, row(b_hh))
    return out


# speedup vs baseline: 1.3673x; 1.2850x over previous
import functools

import jax
import jax.numpy as jnp
from jax import lax
from jax.experimental import pallas as pl

B, L, D = 256, 128, 6
F_ATOM, F_BOND, FP = 39, 10, 128


def _lrelu(x):
    return jnp.where(x >= 0, x, 0.01 * x)


def _dgT(a, b):
    return lax.dot_general(a, b, (((0,), (0,)), ((), ())),
                           preferred_element_type=jnp.float32)


M = 2


def _fused_kernel(atomT_ref, bondT_ref, aidxT_ref, bidxT_ref,
                  w_atom2_ref, wnb_a2_ref, wnb_b2_ref,
                  w1_ref, w2_ref, b_align_ref, w_att_ref, b_att_ref,
                  w_ihT_ref, b_ih_ref, w_hhT_ref, b_hh_ref,
                  out_ref):
    dot = functools.partial(jnp.dot, preferred_element_type=jnp.float32)
    iota_s = lax.broadcasted_iota(jnp.int32, (2 * L, L), 0)
    w2 = w2_ref[...]

    afTs, align1s, tpTs = [], [], []
    for m in range(M):
        atomsT = atomT_ref[m]
        bondsT = bondT_ref[m]
        afT = _lrelu(_dgT(w_atom2_ref[...], atomsT))
        align1 = _dgT(w1_ref[...], afT) + b_align_ref[0, 0]
        apT = _dgT(wnb_a2_ref[...], atomsT)
        bpT = _dgT(wnb_b2_ref[...], bondsT)
        afTs.append(afT)
        align1s.append(align1)
        tpTs.append(jnp.concatenate([apT, bpT], axis=1))

    nfs = [[] for _ in range(M)]
    scores = [[] for _ in range(M)]
    valids = [[] for _ in range(M)]
    for d in range(D):
        for m in range(M):
            arow = aidxT_ref[m, d:d + 1, :]
            brow = bidxT_ref[m, d:d + 1, :]
            g = ((arow == iota_s) | (brow + L == iota_s)).astype(jnp.float32)
            nfT = _lrelu(dot(tpTs[m], g))
            score = _lrelu(align1s[m] + _dgT(w2, nfT))
            valid = (arow != L - 1)
            score = jnp.where(valid, score, score - 9e8)
            nfs[m].append(nfT)
            scores[m].append(score)
            valids[m].append(valid.astype(jnp.float32))

    ctxTs = []
    for m in range(M):
        smax = scores[m][0]
        for d in range(1, D):
            smax = jnp.maximum(smax, scores[m][d])
        exps = [jnp.exp(s - smax) for s in scores[m]]
        denom = exps[0]
        for d in range(1, D):
            denom = denom + exps[d]
        inv = 1.0 / denom
        acc = None
        tot = None
        for d in range(D):
            attn = exps[d] * inv * valids[m][d]
            term = attn * nfs[m][d]
            acc = term if acc is None else acc + term
            tot = attn if tot is None else tot + attn
        ctx_pre = _dgT(w_att_ref[...], acc) + b_att_ref[...] * tot
        ctxTs.append(jnp.where(ctx_pre > 0, ctx_pre, jnp.exp(ctx_pre) - 1.0))

    for m in range(M):
        gi = _dgT(ctxTs[m], w_ihT_ref[...]) + b_ih_ref[...]
        gh = _dgT(afTs[m], w_hhT_ref[...]) + b_hh_ref[...]
        r = jax.nn.sigmoid(gi[:, :FP] + gh[:, :FP])
        z = jax.nn.sigmoid(gi[:, FP:2 * FP] + gh[:, FP:2 * FP])
        n = jnp.tanh(gi[:, 2 * FP:] + r * gh[:, 2 * FP:])
        af = afTs[m].T
        hnew = (1.0 - z) * n + z * af
        out_ref[m] = jnp.maximum(hnew, 0.0)


def kernel(atom_list, bond_list, atom_degree_list, bond_degree_list, atom_mask,
           W_atom, b_atom, W_nb, b_nb, W_align, b_align, W_att, b_att,
           W_ih, W_hh, b_ih, b_hh):
    del atom_mask
    ones = jnp.ones((B, 1, L), jnp.float32)
    atomsT2 = jnp.concatenate([atom_list.transpose(0, 2, 1), ones], axis=1)
    bondsT2 = jnp.concatenate([bond_list.transpose(0, 2, 1), ones], axis=1)
    aidxT = atom_degree_list.astype(jnp.int32).transpose(0, 2, 1)
    bidxT = bond_degree_list.astype(jnp.int32).transpose(0, 2, 1)

    w_atom2 = jnp.concatenate([W_atom, b_atom[None, :]], axis=0)
    half_bnb = 0.5 * b_nb[None, :]
    wnb_a2 = jnp.concatenate([W_nb[:F_ATOM], half_bnb], axis=0)
    wnb_b2 = jnp.concatenate([W_nb[F_ATOM:], half_bnb], axis=0)
    w1 = W_align[:FP]
    w2 = W_align[FP:]
    b_align2 = b_align.reshape(1, 1)
    b_att_col = b_att.reshape(FP, 1)
    w_ihT = W_ih.T
    w_hhT = W_hh.T

    rep = lambda arr: pl.BlockSpec(arr.shape, lambda i: (0,) * arr.ndim)
    row = lambda v: v.reshape(1, -1)

    out = pl.pallas_call(
        _fused_kernel,
        grid=(B // M,),
        in_specs=[
            pl.BlockSpec((M, F_ATOM + 1, L), lambda i: (i, 0, 0)),
            pl.BlockSpec((M, F_BOND + 1, L), lambda i: (i, 0, 0)),
            pl.BlockSpec((M, D, L), lambda i: (i, 0, 0)),
            pl.BlockSpec((M, D, L), lambda i: (i, 0, 0)),
            rep(w_atom2), rep(wnb_a2), rep(wnb_b2),
            rep(w1), rep(w2), rep(b_align2),
            rep(W_att), rep(b_att_col),
            rep(w_ihT), rep(row(b_ih)),
            rep(w_hhT), rep(row(b_hh)),
        ],
        out_specs=pl.BlockSpec((M, L, FP), lambda i: (i, 0, 0)),
        out_shape=jax.ShapeDtypeStruct((B, L, FP), jnp.float32),
    )(atomsT2, bondsT2, aidxT, bidxT,
      w_atom2, wnb_a2, wnb_b2, w1, w2, b_align2, W_att, b_att_col,
      w_ihT, row(b_ih), w_hhT, row(b_hh))
    return out


# speedup vs baseline: 1.5596x; 1.1406x over previous
import functools

import jax
import jax.numpy as jnp
from jax import lax
from jax.experimental import pallas as pl

B, L, D = 256, 128, 6
F_ATOM, F_BOND, FP = 39, 10, 128


def _lrelu(x):
    return jnp.where(x >= 0, x, 0.01 * x)


def _dgT(a, b):
    return lax.dot_general(a, b, (((0,), (0,)), ((), ())),
                           preferred_element_type=jnp.float32)


M = 4


def _fused_kernel(atomT_ref, bondT_ref, aidxT_ref, bidxT_ref,
                  w_atom2_ref, wnb_a2_ref, wnb_b2_ref,
                  w1_ref, w2_ref, b_align_ref, w_att_ref, b_att_ref,
                  w_ihT_ref, b_ih_ref, w_hhT_ref, b_hh_ref,
                  out_ref):
    dot = functools.partial(jnp.dot, preferred_element_type=jnp.float32)
    iota_s = lax.broadcasted_iota(jnp.int32, (2 * L, L), 0)
    w2 = w2_ref[...]

    afTs, align1s, tpTs = [], [], []
    for m in range(M):
        atomsT = atomT_ref[m]
        bondsT = bondT_ref[m]
        afT = _lrelu(_dgT(w_atom2_ref[...], atomsT))
        align1 = _dgT(w1_ref[...], afT) + b_align_ref[0, 0]
        apT = _dgT(wnb_a2_ref[...], atomsT)
        bpT = _dgT(wnb_b2_ref[...], bondsT)
        afTs.append(afT)
        align1s.append(align1)
        tpTs.append(jnp.concatenate([apT, bpT], axis=1))

    nfs = [[] for _ in range(M)]
    scores = [[] for _ in range(M)]
    valids = [[] for _ in range(M)]
    for d in range(D):
        for m in range(M):
            arow = aidxT_ref[m, d:d + 1, :]
            brow = bidxT_ref[m, d:d + 1, :]
            g = ((arow == iota_s) | (brow + L == iota_s)).astype(jnp.float32)
            nfT = _lrelu(dot(tpTs[m], g))
            score = _lrelu(align1s[m] + _dgT(w2, nfT))
            valid = (arow != L - 1)
            score = jnp.where(valid, score, score - 9e8)
            nfs[m].append(nfT)
            scores[m].append(score)
            valids[m].append(valid.astype(jnp.float32))

    ctxTs = []
    for m in range(M):
        smax = scores[m][0]
        for d in range(1, D):
            smax = jnp.maximum(smax, scores[m][d])
        exps = [jnp.exp(s - smax) for s in scores[m]]
        denom = exps[0]
        for d in range(1, D):
            denom = denom + exps[d]
        inv = 1.0 / denom
        acc = None
        tot = None
        for d in range(D):
            attn = exps[d] * inv * valids[m][d]
            term = attn * nfs[m][d]
            acc = term if acc is None else acc + term
            tot = attn if tot is None else tot + attn
        ctx_pre = _dgT(w_att_ref[...], acc) + b_att_ref[...] * tot
        ctxTs.append(jnp.where(ctx_pre > 0, ctx_pre, jnp.exp(ctx_pre) - 1.0))

    for m in range(M):
        gi = _dgT(ctxTs[m], w_ihT_ref[...]) + b_ih_ref[...]
        gh = _dgT(afTs[m], w_hhT_ref[...]) + b_hh_ref[...]
        r = jax.nn.sigmoid(gi[:, :FP] + gh[:, :FP])
        z = jax.nn.sigmoid(gi[:, FP:2 * FP] + gh[:, FP:2 * FP])
        n = jnp.tanh(gi[:, 2 * FP:] + r * gh[:, 2 * FP:])
        af = afTs[m].T
        hnew = (1.0 - z) * n + z * af
        out_ref[m] = jnp.maximum(hnew, 0.0)


def kernel(atom_list, bond_list, atom_degree_list, bond_degree_list, atom_mask,
           W_atom, b_atom, W_nb, b_nb, W_align, b_align, W_att, b_att,
           W_ih, W_hh, b_ih, b_hh):
    del atom_mask
    ones = jnp.ones((B, 1, L), jnp.float32)
    atomsT2 = jnp.concatenate([atom_list.transpose(0, 2, 1), ones], axis=1)
    bondsT2 = jnp.concatenate([bond_list.transpose(0, 2, 1), ones], axis=1)
    aidxT = atom_degree_list.astype(jnp.int32).transpose(0, 2, 1)
    bidxT = bond_degree_list.astype(jnp.int32).transpose(0, 2, 1)

    w_atom2 = jnp.concatenate([W_atom, b_atom[None, :]], axis=0)
    half_bnb = 0.5 * b_nb[None, :]
    wnb_a2 = jnp.concatenate([W_nb[:F_ATOM], half_bnb], axis=0)
    wnb_b2 = jnp.concatenate([W_nb[F_ATOM:], half_bnb], axis=0)
    w1 = W_align[:FP]
    w2 = W_align[FP:]
    b_align2 = b_align.reshape(1, 1)
    b_att_col = b_att.reshape(FP, 1)
    w_ihT = W_ih.T
    w_hhT = W_hh.T

    rep = lambda arr: pl.BlockSpec(arr.shape, lambda i: (0,) * arr.ndim)
    row = lambda v: v.reshape(1, -1)

    out = pl.pallas_call(
        _fused_kernel,
        grid=(B // M,),
        in_specs=[
            pl.BlockSpec((M, F_ATOM + 1, L), lambda i: (i, 0, 0)),
            pl.BlockSpec((M, F_BOND + 1, L), lambda i: (i, 0, 0)),
            pl.BlockSpec((M, D, L), lambda i: (i, 0, 0)),
            pl.BlockSpec((M, D, L), lambda i: (i, 0, 0)),
            rep(w_atom2), rep(wnb_a2), rep(wnb_b2),
            rep(w1), rep(w2), rep(b_align2),
            rep(W_att), rep(b_att_col),
            rep(w_ihT), rep(row(b_ih)),
            rep(w_hhT), rep(row(b_hh)),
        ],
        out_specs=pl.BlockSpec((M, L, FP), lambda i: (i, 0, 0)),
        out_shape=jax.ShapeDtypeStruct((B, L, FP), jnp.float32),
    )(atomsT2, bondsT2, aidxT, bidxT,
      w_atom2, wnb_a2, wnb_b2, w1, w2, b_align2, W_att, b_att_col,
      w_ihT, row(b_ih), w_hhT, row(b_hh))
    return out


# speedup vs baseline: 3.1205x; 2.0008x over previous
import functools

import jax
import jax.numpy as jnp
from jax import lax
from jax.experimental import pallas as pl

B, L, D = 256, 128, 6
F_ATOM, F_BOND, FP = 39, 10, 128


def _lrelu(x):
    return jnp.where(x >= 0, x, 0.01 * x)


def _dgT(a, b):
    return lax.dot_general(a, b, (((0,), (0,)), ((), ())),
                           preferred_element_type=jnp.float32)


M = 4


def _fused_kernel(atomT_ref, bondT_ref, aidxT_ref, bidxT_ref,
                  w_atom2_ref, wnb_a2_ref, wnb_b2_ref,
                  w1_ref, w2_ref, b_align_ref, w_att_ref, b_att_ref,
                  w_ihT_ref, b_ih_ref, w_hhT_ref, b_hh_ref,
                  out_ref):
    dot = functools.partial(jnp.dot, preferred_element_type=jnp.float32)
    iota_s = lax.broadcasted_iota(jnp.int32, (2 * L, L), 0)
    w2 = w2_ref[...]

    afTs, af16s, align1s, tpTs = [], [], [], []
    for m in range(M):
        atomsT = atomT_ref[m]
        bondsT = bondT_ref[m]
        afT = _lrelu(_dgT(w_atom2_ref[...], atomsT))
        af16 = afT.astype(jnp.bfloat16)
        align1 = _dgT(w1_ref[...], af16) + b_align_ref[0, 0]
        apT = _dgT(wnb_a2_ref[...], atomsT)
        bpT = _dgT(wnb_b2_ref[...], bondsT)
        afTs.append(afT)
        af16s.append(af16)
        align1s.append(align1)
        tpTs.append(jnp.concatenate([apT, bpT], axis=1).astype(jnp.bfloat16))

    nf_alls, score_alls, valids = [], [], []
    for m in range(M):
        gcols = []
        for d in range(D):
            arow = aidxT_ref[m, d:d + 1, :]
            brow = bidxT_ref[m, d:d + 1, :]
            gcols.append(((arow == iota_s)
                          | (brow + L == iota_s)).astype(jnp.bfloat16))
        g_all = jnp.concatenate(gcols, axis=1)
        nf_all = _lrelu(dot(tpTs[m], g_all)).astype(jnp.bfloat16)
        sc_all = _dgT(w2, nf_all)
        nf_alls.append(nf_all)
        score_alls.append(sc_all)
        valids.append((aidxT_ref[m] != L - 1).astype(jnp.float32))

    ctxTs = []
    for m in range(M):
        al = align1s[m]
        scores = []
        for d in range(D):
            s = _lrelu(al + score_alls[m][:, d * L:(d + 1) * L])
            v = valids[m][d:d + 1, :]
            scores.append(jnp.where(v > 0, s, s - 9e8))
        smax = scores[0]
        for d in range(1, D):
            smax = jnp.maximum(smax, scores[d])
        exps = [jnp.exp(s - smax) for s in scores]
        denom = exps[0]
        for d in range(1, D):
            denom = denom + exps[d]
        inv = 1.0 / denom
        acc = None
        tot = None
        for d in range(D):
            attn = exps[d] * inv * valids[m][d:d + 1, :]
            term = attn * nf_alls[m][:, d * L:(d + 1) * L]
            acc = term if acc is None else acc + term
            tot = attn if tot is None else tot + attn
        ctx_pre = (_dgT(w_att_ref[...], acc.astype(jnp.bfloat16))
                   + b_att_ref[...] * tot)
        ctx = jnp.where(ctx_pre > 0, ctx_pre, jnp.exp(ctx_pre) - 1.0)
        ctxTs.append(ctx.astype(jnp.bfloat16))

    for m in range(M):
        gi = _dgT(ctxTs[m], w_ihT_ref[...]) + b_ih_ref[...]
        gh = _dgT(af16s[m], w_hhT_ref[...]) + b_hh_ref[...]
        r = jax.nn.sigmoid(gi[:, :FP] + gh[:, :FP])
        z = jax.nn.sigmoid(gi[:, FP:2 * FP] + gh[:, FP:2 * FP])
        n = jnp.tanh(gi[:, 2 * FP:] + r * gh[:, 2 * FP:])
        af = afTs[m].T
        hnew = (1.0 - z) * n + z * af
        out_ref[m] = jnp.maximum(hnew, 0.0)


def kernel(atom_list, bond_list, atom_degree_list, bond_degree_list, atom_mask,
           W_atom, b_atom, W_nb, b_nb, W_align, b_align, W_att, b_att,
           W_ih, W_hh, b_ih, b_hh):
    del atom_mask
    ones = jnp.ones((B, 1, L), jnp.float32)
    bf = jnp.bfloat16
    atomsT2 = jnp.concatenate([atom_list.transpose(0, 2, 1), ones], axis=1).astype(bf)
    bondsT2 = jnp.concatenate([bond_list.transpose(0, 2, 1), ones], axis=1).astype(bf)
    aidxT = atom_degree_list.astype(jnp.int32).transpose(0, 2, 1)
    bidxT = bond_degree_list.astype(jnp.int32).transpose(0, 2, 1)

    w_atom2 = jnp.concatenate([W_atom, b_atom[None, :]], axis=0).astype(bf)
    half_bnb = 0.5 * b_nb[None, :]
    wnb_a2 = jnp.concatenate([W_nb[:F_ATOM], half_bnb], axis=0).astype(bf)
    wnb_b2 = jnp.concatenate([W_nb[F_ATOM:], half_bnb], axis=0).astype(bf)
    w1 = W_align[:FP].astype(bf)
    w2 = W_align[FP:].astype(bf)
    b_align2 = b_align.reshape(1, 1)
    b_att_col = b_att.reshape(FP, 1)
    w_att16 = W_att.astype(bf)
    w_ihT = W_ih.T.astype(bf)
    w_hhT = W_hh.T.astype(bf)

    rep = lambda arr: pl.BlockSpec(arr.shape, lambda i: (0,) * arr.ndim)
    row = lambda v: v.reshape(1, -1)

    out = pl.pallas_call(
        _fused_kernel,
        grid=(B // M,),
        in_specs=[
            pl.BlockSpec((M, F_ATOM + 1, L), lambda i: (i, 0, 0)),
            pl.BlockSpec((M, F_BOND + 1, L), lambda i: (i, 0, 0)),
            pl.BlockSpec((M, D, L), lambda i: (i, 0, 0)),
            pl.BlockSpec((M, D, L), lambda i: (i, 0, 0)),
            rep(w_atom2), rep(wnb_a2), rep(wnb_b2),
            rep(w1), rep(w2), rep(b_align2),
            rep(w_att16), rep(b_att_col),
            rep(w_ihT), rep(row(b_ih)),
            rep(w_hhT), rep(row(b_hh)),
        ],
        out_specs=pl.BlockSpec((M, L, FP), lambda i: (i, 0, 0)),
        out_shape=jax.ShapeDtypeStruct((B, L, FP), jnp.float32),
    )(atomsT2, bondsT2, aidxT, bidxT,
      w_atom2, wnb_a2, wnb_b2, w1, w2, b_align2, w_att16, b_att_col,
      w_ihT, row(b_ih), w_hhT, row(b_hh))
    return out


# speedup vs baseline: 3.3593x; 1.0765x over previous
import functools

import jax
import jax.numpy as jnp
from jax import lax
from jax.experimental import pallas as pl

B, L, D = 256, 128, 6
F_ATOM, F_BOND, FP = 39, 10, 128


def _lrelu(x):
    return jnp.where(x >= 0, x, 0.01 * x)


def _dgT(a, b):
    return lax.dot_general(a, b, (((0,), (0,)), ((), ())),
                           preferred_element_type=jnp.float32)


M = 8


def _fused_kernel(atomT_ref, bondT_ref, aidxT_ref, bidxT_ref,
                  w_atom2_ref, wnb_a2_ref, wnb_b2_ref,
                  w1_ref, w2_ref, b_align_ref, w_att_ref, b_att_ref,
                  w_ihT_ref, b_ih_ref, w_hhT_ref, b_hh_ref,
                  out_ref):
    dot = functools.partial(jnp.dot, preferred_element_type=jnp.float32)
    iota_s = lax.broadcasted_iota(jnp.int32, (2 * L, L), 0)
    w2 = w2_ref[...]

    afTs, af16s, align1s, tpTs = [], [], [], []
    for m in range(M):
        atomsT = atomT_ref[m]
        bondsT = bondT_ref[m]
        afT = _lrelu(_dgT(w_atom2_ref[...], atomsT))
        af16 = afT.astype(jnp.bfloat16)
        align1 = _dgT(w1_ref[...], af16) + b_align_ref[0, 0]
        apT = _dgT(wnb_a2_ref[...], atomsT)
        bpT = _dgT(wnb_b2_ref[...], bondsT)
        afTs.append(afT)
        af16s.append(af16)
        align1s.append(align1)
        tpTs.append(jnp.concatenate([apT, bpT], axis=1).astype(jnp.bfloat16))

    nf_alls, score_alls, valids = [], [], []
    for m in range(M):
        gcols = []
        for d in range(D):
            arow = aidxT_ref[m, d:d + 1, :]
            brow = bidxT_ref[m, d:d + 1, :]
            gcols.append(((arow == iota_s)
                          | (brow + L == iota_s)).astype(jnp.bfloat16))
        g_all = jnp.concatenate(gcols, axis=1)
        nf_all = _lrelu(dot(tpTs[m], g_all)).astype(jnp.bfloat16)
        sc_all = _dgT(w2, nf_all)
        nf_alls.append(nf_all)
        score_alls.append(sc_all)
        valids.append((aidxT_ref[m] != L - 1).astype(jnp.float32))

    ctxTs = []
    for m in range(M):
        al = align1s[m]
        scores = []
        for d in range(D):
            s = _lrelu(al + score_alls[m][:, d * L:(d + 1) * L])
            v = valids[m][d:d + 1, :]
            scores.append(jnp.where(v > 0, s, s - 9e8))
        smax = scores[0]
        for d in range(1, D):
            smax = jnp.maximum(smax, scores[d])
        exps = [jnp.exp(s - smax) for s in scores]
        denom = exps[0]
        for d in range(1, D):
            denom = denom + exps[d]
        inv = 1.0 / denom
        acc = None
        tot = None
        for d in range(D):
            attn = exps[d] * inv * valids[m][d:d + 1, :]
            term = attn * nf_alls[m][:, d * L:(d + 1) * L]
            acc = term if acc is None else acc + term
            tot = attn if tot is None else tot + attn
        ctx_pre = (_dgT(w_att_ref[...], acc.astype(jnp.bfloat16))
                   + b_att_ref[...] * tot)
        ctx = jnp.where(ctx_pre > 0, ctx_pre, jnp.exp(ctx_pre) - 1.0)
        ctxTs.append(ctx.astype(jnp.bfloat16))

    for m in range(M):
        gi = _dgT(ctxTs[m], w_ihT_ref[...]) + b_ih_ref[...]
        gh = _dgT(af16s[m], w_hhT_ref[...]) + b_hh_ref[...]
        r = jax.nn.sigmoid(gi[:, :FP] + gh[:, :FP])
        z = jax.nn.sigmoid(gi[:, FP:2 * FP] + gh[:, FP:2 * FP])
        n = jnp.tanh(gi[:, 2 * FP:] + r * gh[:, 2 * FP:])
        af = afTs[m].T
        hnew = (1.0 - z) * n + z * af
        out_ref[m] = jnp.maximum(hnew, 0.0)


def kernel(atom_list, bond_list, atom_degree_list, bond_degree_list, atom_mask,
           W_atom, b_atom, W_nb, b_nb, W_align, b_align, W_att, b_att,
           W_ih, W_hh, b_ih, b_hh):
    del atom_mask
    ones = jnp.ones((B, 1, L), jnp.float32)
    bf = jnp.bfloat16
    atomsT2 = jnp.concatenate([atom_list.transpose(0, 2, 1), ones], axis=1).astype(bf)
    bondsT2 = jnp.concatenate([bond_list.transpose(0, 2, 1), ones], axis=1).astype(bf)
    aidxT = atom_degree_list.astype(jnp.int32).transpose(0, 2, 1)
    bidxT = bond_degree_list.astype(jnp.int32).transpose(0, 2, 1)

    w_atom2 = jnp.concatenate([W_atom, b_atom[None, :]], axis=0).astype(bf)
    half_bnb = 0.5 * b_nb[None, :]
    wnb_a2 = jnp.concatenate([W_nb[:F_ATOM], half_bnb], axis=0).astype(bf)
    wnb_b2 = jnp.concatenate([W_nb[F_ATOM:], half_bnb], axis=0).astype(bf)
    w1 = W_align[:FP].astype(bf)
    w2 = W_align[FP:].astype(bf)
    b_align2 = b_align.reshape(1, 1)
    b_att_col = b_att.reshape(FP, 1)
    w_att16 = W_att.astype(bf)
    w_ihT = W_ih.T.astype(bf)
    w_hhT = W_hh.T.astype(bf)

    rep = lambda arr: pl.BlockSpec(arr.shape, lambda i: (0,) * arr.ndim)
    row = lambda v: v.reshape(1, -1)

    out = pl.pallas_call(
        _fused_kernel,
        grid=(B // M,),
        in_specs=[
            pl.BlockSpec((M, F_ATOM + 1, L), lambda i: (i, 0, 0)),
            pl.BlockSpec((M, F_BOND + 1, L), lambda i: (i, 0, 0)),
            pl.BlockSpec((M, D, L), lambda i: (i, 0, 0)),
            pl.BlockSpec((M, D, L), lambda i: (i, 0, 0)),
            rep(w_atom2), rep(wnb_a2), rep(wnb_b2),
            rep(w1), rep(w2), rep(b_align2),
            rep(w_att16), rep(b_att_col),
            rep(w_ihT), rep(row(b_ih)),
            rep(w_hhT), rep(row(b_hh)),
        ],
        out_specs=pl.BlockSpec((M, L, FP), lambda i: (i, 0, 0)),
        out_shape=jax.ShapeDtypeStruct((B, L, FP), jnp.float32),
    )(atomsT2, bondsT2, aidxT, bidxT,
      w_atom2, wnb_a2, wnb_b2, w1, w2, b_align2, w_att16, b_att_col,
      w_ihT, row(b_ih), w_hhT, row(b_hh))
    return out


# speedup vs baseline: 3.4217x; 1.0186x over previous
import functools

import jax
import jax.numpy as jnp
from jax import lax
from jax.experimental import pallas as pl

B, L, D = 256, 128, 6
F_ATOM, F_BOND, FP = 39, 10, 128


def _lrelu(x):
    return jnp.where(x >= 0, x, 0.01 * x)


def _dgT(a, b):
    return lax.dot_general(a, b, (((0,), (0,)), ((), ())),
                           preferred_element_type=jnp.float32)


M = 8


def _fused_kernel(atomT_ref, bondT_ref, aidxT_ref, bidxT_ref,
                  w_atom2_ref, wnb_a2_ref, wnb_b2_ref,
                  w1_ref, w2_ref, b_align_ref, w_att_ref, b_att_ref,
                  w_ihT_ref, b_ih_ref, w_hhT_ref, b_hh_ref,
                  out_ref):
    dot = functools.partial(jnp.dot, preferred_element_type=jnp.float32)
    iota_b = lax.broadcasted_iota(jnp.int32, (L, L), 0).astype(jnp.bfloat16)
    w2 = w2_ref[...]

    afTs, af16s, align1s, apTs, bpTs = [], [], [], [], []
    for m in range(M):
        atomsT = atomT_ref[m]
        bondsT = bondT_ref[m]
        afT = _lrelu(_dgT(w_atom2_ref[...], atomsT))
        af16 = afT.astype(jnp.bfloat16)
        align1 = _dgT(w1_ref[...], af16) + b_align_ref[0, 0]
        apTs.append(_dgT(wnb_a2_ref[...], atomsT).astype(jnp.bfloat16))
        bpTs.append(_dgT(wnb_b2_ref[...], bondsT).astype(jnp.bfloat16))
        afTs.append(afT)
        af16s.append(af16)
        align1s.append(align1)

    nf_alls, score_alls, valids = [], [], []
    for m in range(M):
        gacols, gbcols = [], []
        for d in range(D):
            arow = aidxT_ref[m, d:d + 1, :]
            brow = bidxT_ref[m, d:d + 1, :]
            gacols.append((arow == iota_b).astype(jnp.bfloat16))
            gbcols.append((brow == iota_b).astype(jnp.bfloat16))
        ga = jnp.concatenate(gacols, axis=1)
        gb = jnp.concatenate(gbcols, axis=1)
        nf_all = _lrelu(dot(apTs[m], ga)
                        + dot(bpTs[m], gb)).astype(jnp.bfloat16)
        sc_all = _dgT(w2, nf_all)
        nf_alls.append(nf_all)
        score_alls.append(sc_all)
        valids.append((aidxT_ref[m] != L - 1).astype(jnp.float32))

    ctxTs = []
    for m in range(M):
        al = align1s[m]
        scores = []
        for d in range(D):
            s = _lrelu(al + score_alls[m][:, d * L:(d + 1) * L])
            v = valids[m][d:d + 1, :]
            scores.append(jnp.where(v > 0, s, s - 9e8))
        smax = scores[0]
        for d in range(1, D):
            smax = jnp.maximum(smax, scores[d])
        exps = [jnp.exp(s - smax) for s in scores]
        denom = exps[0]
        for d in range(1, D):
            denom = denom + exps[d]
        inv = 1.0 / denom
        acc = None
        tot = None
        for d in range(D):
            attn = exps[d] * inv * valids[m][d:d + 1, :]
            term = attn * nf_alls[m][:, d * L:(d + 1) * L]
            acc = term if acc is None else acc + term
            tot = attn if tot is None else tot + attn
        ctx_pre = (_dgT(w_att_ref[...], acc.astype(jnp.bfloat16))
                   + b_att_ref[...] * tot)
        ctx = jnp.where(ctx_pre > 0, ctx_pre, jnp.exp(ctx_pre) - 1.0)
        ctxTs.append(ctx.astype(jnp.bfloat16))

    for m in range(M):
        gi = _dgT(ctxTs[m], w_ihT_ref[...]) + b_ih_ref[...]
        gh = _dgT(af16s[m], w_hhT_ref[...]) + b_hh_ref[...]
        r = jax.nn.sigmoid(gi[:, :FP] + gh[:, :FP])
        z = jax.nn.sigmoid(gi[:, FP:2 * FP] + gh[:, FP:2 * FP])
        n = jnp.tanh(gi[:, 2 * FP:] + r * gh[:, 2 * FP:])
        af = afTs[m].T
        hnew = (1.0 - z) * n + z * af
        out_ref[m] = jnp.maximum(hnew, 0.0)


def kernel(atom_list, bond_list, atom_degree_list, bond_degree_list, atom_mask,
           W_atom, b_atom, W_nb, b_nb, W_align, b_align, W_att, b_att,
           W_ih, W_hh, b_ih, b_hh):
    del atom_mask
    ones = jnp.ones((B, 1, L), jnp.float32)
    bf = jnp.bfloat16
    atomsT2 = jnp.concatenate([atom_list.transpose(0, 2, 1), ones], axis=1).astype(bf)
    bondsT2 = jnp.concatenate([bond_list.transpose(0, 2, 1), ones], axis=1).astype(bf)
    aidxT = atom_degree_list.astype(jnp.int32).transpose(0, 2, 1).astype(bf)
    bidxT = bond_degree_list.astype(jnp.int32).transpose(0, 2, 1).astype(bf)

    w_atom2 = jnp.concatenate([W_atom, b_atom[None, :]], axis=0).astype(bf)
    half_bnb = 0.5 * b_nb[None, :]
    wnb_a2 = jnp.concatenate([W_nb[:F_ATOM], half_bnb], axis=0).astype(bf)
    wnb_b2 = jnp.concatenate([W_nb[F_ATOM:], half_bnb], axis=0).astype(bf)
    w1 = W_align[:FP].astype(bf)
    w2 = W_align[FP:].astype(bf)
    b_align2 = b_align.reshape(1, 1)
    b_att_col = b_att.reshape(FP, 1)
    w_att16 = W_att.astype(bf)
    w_ihT = W_ih.T.astype(bf)
    w_hhT = W_hh.T.astype(bf)

    rep = lambda arr: pl.BlockSpec(arr.shape, lambda i: (0,) * arr.ndim)
    row = lambda v: v.reshape(1, -1)

    out = pl.pallas_call(
        _fused_kernel,
        grid=(B // M,),
        in_specs=[
            pl.BlockSpec((M, F_ATOM + 1, L), lambda i: (i, 0, 0)),
            pl.BlockSpec((M, F_BOND + 1, L), lambda i: (i, 0, 0)),
            pl.BlockSpec((M, D, L), lambda i: (i, 0, 0)),
            pl.BlockSpec((M, D, L), lambda i: (i, 0, 0)),
            rep(w_atom2), rep(wnb_a2), rep(wnb_b2),
            rep(w1), rep(w2), rep(b_align2),
            rep(w_att16), rep(b_att_col),
            rep(w_ihT), rep(row(b_ih)),
            rep(w_hhT), rep(row(b_hh)),
        ],
        out_specs=pl.BlockSpec((M, L, FP), lambda i: (i, 0, 0)),
        out_shape=jax.ShapeDtypeStruct((B, L, FP), jnp.float32),
    )(atomsT2, bondsT2, aidxT, bidxT,
      w_atom2, wnb_a2, wnb_b2, w1, w2, b_align2, w_att16, b_att_col,
      w_ihT, row(b_ih), w_hhT, row(b_hh))
    return out


# speedup vs baseline: 3.5674x; 1.0426x over previous
import functools

import jax
import jax.numpy as jnp
from jax import lax
from jax.experimental import pallas as pl

B, L, D = 256, 128, 6
F_ATOM, F_BOND, FP = 39, 10, 128


def _lrelu(x):
    return jnp.where(x >= 0, x, 0.01 * x)


def _dgT(a, b):
    return lax.dot_general(a, b, (((0,), (0,)), ((), ())),
                           preferred_element_type=jnp.float32)


M = 8


def _fused_kernel(atomT_ref, bondT_ref, aidxT_ref, bidxT_ref,
                  w_atom2_ref, wnb_a2_ref, wnb_b2_ref,
                  w1_ref, w2_ref, b_align_ref, w_att_ref, b_att_ref,
                  w_ihT_ref, b_ih_ref, w_hhT_ref, b_hh_ref,
                  out_ref):
    dot = functools.partial(jnp.dot, preferred_element_type=jnp.float32)
    iota_b = lax.broadcasted_iota(jnp.int32, (L, L), 0).astype(jnp.bfloat16)
    w2 = w2_ref[...]

    afTs, af16s, align1s, apTs, bpTs = [], [], [], [], []
    for m in range(M):
        atomsT = atomT_ref[m]
        bondsT = bondT_ref[m]
        afT = _lrelu(_dgT(w_atom2_ref[...], atomsT))
        af16 = afT.astype(jnp.bfloat16)
        align1 = _dgT(w1_ref[...], af16) + b_align_ref[0, 0]
        apTs.append(_dgT(wnb_a2_ref[...], atomsT).astype(jnp.bfloat16))
        bpTs.append(_dgT(wnb_b2_ref[...], bondsT).astype(jnp.bfloat16))
        afTs.append(afT)
        af16s.append(af16)
        align1s.append(align1)

    nf_alls, score_alls, valids = [], [], []
    for m in range(M):
        gacols, gbcols = [], []
        for d in range(D):
            arow = aidxT_ref[m, d:d + 1, :]
            brow = bidxT_ref[m, d:d + 1, :]
            da = arow - iota_b
            db = brow - iota_b
            gacols.append(jnp.maximum(1.0 - da * da, 0.0))
            gbcols.append(jnp.maximum(1.0 - db * db, 0.0))
        ga = jnp.concatenate(gacols, axis=1)
        gb = jnp.concatenate(gbcols, axis=1)
        nf_all = _lrelu(dot(apTs[m], ga)
                        + dot(bpTs[m], gb)).astype(jnp.bfloat16)
        sc_all = _dgT(w2, nf_all)
        nf_alls.append(nf_all)
        score_alls.append(sc_all)
        valids.append((aidxT_ref[m] != L - 1).astype(jnp.float32))

    ctxTs = []
    for m in range(M):
        al = align1s[m]
        scores = []
        for d in range(D):
            s = _lrelu(al + score_alls[m][:, d * L:(d + 1) * L])
            v = valids[m][d:d + 1, :]
            scores.append(jnp.where(v > 0, s, s - 9e8))
        smax = scores[0]
        for d in range(1, D):
            smax = jnp.maximum(smax, scores[d])
        exps = [jnp.exp(s - smax) for s in scores]
        denom = exps[0]
        for d in range(1, D):
            denom = denom + exps[d]
        inv = 1.0 / denom
        acc = None
        tot = None
        for d in range(D):
            attn = exps[d] * inv * valids[m][d:d + 1, :]
            term = attn * nf_alls[m][:, d * L:(d + 1) * L]
            acc = term if acc is None else acc + term
            tot = attn if tot is None else tot + attn
        ctx_pre = (_dgT(w_att_ref[...], acc.astype(jnp.bfloat16))
                   + b_att_ref[...] * tot)
        ctx = jnp.where(ctx_pre > 0, ctx_pre, jnp.exp(ctx_pre) - 1.0)
        ctxTs.append(ctx.astype(jnp.bfloat16))

    for m in range(M):
        gi = _dgT(ctxTs[m], w_ihT_ref[...]) + b_ih_ref[...]
        gh = _dgT(af16s[m], w_hhT_ref[...]) + b_hh_ref[...]
        r = jax.nn.sigmoid(gi[:, :FP] + gh[:, :FP])
        z = jax.nn.sigmoid(gi[:, FP:2 * FP] + gh[:, FP:2 * FP])
        n = jnp.tanh(gi[:, 2 * FP:] + r * gh[:, 2 * FP:])
        af = afTs[m].T
        hnew = (1.0 - z) * n + z * af
        out_ref[m] = jnp.maximum(hnew, 0.0)


def kernel(atom_list, bond_list, atom_degree_list, bond_degree_list, atom_mask,
           W_atom, b_atom, W_nb, b_nb, W_align, b_align, W_att, b_att,
           W_ih, W_hh, b_ih, b_hh):
    del atom_mask
    ones = jnp.ones((B, 1, L), jnp.float32)
    bf = jnp.bfloat16
    atomsT2 = jnp.concatenate([atom_list.transpose(0, 2, 1), ones], axis=1).astype(bf)
    bondsT2 = jnp.concatenate([bond_list.transpose(0, 2, 1), ones], axis=1).astype(bf)
    aidxT = atom_degree_list.astype(jnp.int32).transpose(0, 2, 1).astype(bf)
    bidxT = bond_degree_list.astype(jnp.int32).transpose(0, 2, 1).astype(bf)

    w_atom2 = jnp.concatenate([W_atom, b_atom[None, :]], axis=0).astype(bf)
    half_bnb = 0.5 * b_nb[None, :]
    wnb_a2 = jnp.concatenate([W_nb[:F_ATOM], half_bnb], axis=0).astype(bf)
    wnb_b2 = jnp.concatenate([W_nb[F_ATOM:], half_bnb], axis=0).astype(bf)
    w1 = W_align[:FP].astype(bf)
    w2 = W_align[FP:].astype(bf)
    b_align2 = b_align.reshape(1, 1)
    b_att_col = b_att.reshape(FP, 1)
    w_att16 = W_att.astype(bf)
    w_ihT = W_ih.T.astype(bf)
    w_hhT = W_hh.T.astype(bf)

    rep = lambda arr: pl.BlockSpec(arr.shape, lambda i: (0,) * arr.ndim)
    row = lambda v: v.reshape(1, -1)

    out = pl.pallas_call(
        _fused_kernel,
        grid=(B // M,),
        in_specs=[
            pl.BlockSpec((M, F_ATOM + 1, L), lambda i: (i, 0, 0)),
            pl.BlockSpec((M, F_BOND + 1, L), lambda i: (i, 0, 0)),
            pl.BlockSpec((M, D, L), lambda i: (i, 0, 0)),
            pl.BlockSpec((M, D, L), lambda i: (i, 0, 0)),
            rep(w_atom2), rep(wnb_a2), rep(wnb_b2),
            rep(w1), rep(w2), rep(b_align2),
            rep(w_att16), rep(b_att_col),
            rep(w_ihT), rep(row(b_ih)),
            rep(w_hhT), rep(row(b_hh)),
        ],
        out_specs=pl.BlockSpec((M, L, FP), lambda i: (i, 0, 0)),
        out_shape=jax.ShapeDtypeStruct((B, L, FP), jnp.float32),
    )(atomsT2, bondsT2, aidxT, bidxT,
      w_atom2, wnb_a2, wnb_b2, w1, w2, b_align2, w_att16, b_att_col,
      w_ihT, row(b_ih), w_hhT, row(b_hh))
    return out


# speedup vs baseline: 4.2782x; 1.1992x over previous
import functools

import jax
import jax.numpy as jnp
from jax import lax
from jax.experimental import pallas as pl

B, L, D = 256, 128, 6
F_ATOM, F_BOND, FP = 39, 10, 128


def _lrelu(x):
    return jnp.maximum(x, 0.01 * x)


def _dgT(a, b):
    return lax.dot_general(a, b, (((0,), (0,)), ((), ())),
                           preferred_element_type=jnp.float32)


M = 8


def _fused_kernel(atomT_ref, bondT_ref, aidxT_ref, bidxT_ref,
                  w_atom2_ref, wnb_a2_ref, wnb_b2_ref,
                  w1_ref, w2_ref, b_align_ref, w_att_ref, b_att_ref,
                  w_ihT_ref, b_ih_ref, w_hhT_ref, b_hh_ref,
                  out_ref):
    dot = functools.partial(jnp.dot, preferred_element_type=jnp.float32)
    bf = jnp.bfloat16
    iota_b = lax.broadcasted_iota(jnp.int32, (L, L), 0).astype(bf)
    w2 = w2_ref[...]
    ML = M * L

    atomsT_all = jnp.concatenate([atomT_ref[m] for m in range(M)], axis=1)
    bondsT_all = jnp.concatenate([bondT_ref[m] for m in range(M)], axis=1)
    afT_all = _lrelu(_dgT(w_atom2_ref[...], atomsT_all))
    af16_all = afT_all.astype(bf)
    align1_all = _dgT(w1_ref[...], af16_all) + b_align_ref[0, 0]
    apT_all = _dgT(wnb_a2_ref[...], atomsT_all).astype(bf)
    bpT_all = _dgT(wnb_b2_ref[...], bondsT_all).astype(bf)

    nf_alls, score_alls, valids = [], [], []
    for m in range(M):
        gacols, gbcols = [], []
        for d in range(D):
            arow = aidxT_ref[m, d:d + 1, :]
            brow = bidxT_ref[m, d:d + 1, :]
            da = arow - iota_b
            db = brow - iota_b
            gacols.append(jnp.maximum(1.0 - da * da, 0.0))
            gbcols.append(jnp.maximum(1.0 - db * db, 0.0))
        g2 = jnp.concatenate([jnp.concatenate(gacols, axis=1),
                              jnp.concatenate(gbcols, axis=1)],
                             axis=0)
        tp = jnp.concatenate([apT_all[:, m * L:(m + 1) * L],
                              bpT_all[:, m * L:(m + 1) * L]], axis=1)
        nf16 = _lrelu(dot(tp, g2).astype(bf))
        sc_all = _dgT(w2, nf16)
        nf_alls.append(nf16)
        score_alls.append(sc_all)
        valids.append((aidxT_ref[m] != L - 1).astype(jnp.float32))

    accs, tots = [], []
    for m in range(M):
        al = align1_all[:, m * L:(m + 1) * L]
        scores = []
        for d in range(D):
            s = _lrelu(al + score_alls[m][:, d * L:(d + 1) * L])
            v = valids[m][d:d + 1, :]
            scores.append(jnp.where(v > 0, s, s - 9e8))
        smax = scores[0]
        for d in range(1, D):
            smax = jnp.maximum(smax, scores[d])
        exps = [jnp.exp(s - smax) for s in scores]
        denom = exps[0]
        for d in range(1, D):
            denom = denom + exps[d]
        inv = 1.0 / denom
        acc = None
        tot = None
        for d in range(D):
            attn = exps[d] * inv * valids[m][d:d + 1, :]
            term = attn * nf_alls[m][:, d * L:(d + 1) * L]
            acc = term if acc is None else acc + term
            tot = attn if tot is None else tot + attn
        accs.append(acc.astype(bf))
        tots.append(tot)
    acc_all = jnp.concatenate(accs, axis=1)
    tot_all = jnp.concatenate(tots, axis=1)
    ctx_pre = (_dgT(w_att_ref[...], acc_all)
               + b_att_ref[...] * tot_all)
    ctx16 = jnp.where(ctx_pre > 0, ctx_pre,
                      jnp.exp(ctx_pre) - 1.0).astype(bf)

    gi = _dgT(ctx16, w_ihT_ref[...]) + b_ih_ref[...]
    gh = _dgT(af16_all, w_hhT_ref[...]) + b_hh_ref[...]
    r = jax.nn.sigmoid(gi[:, :FP] + gh[:, :FP])
    z = jax.nn.sigmoid(gi[:, FP:2 * FP] + gh[:, FP:2 * FP])
    n = jnp.tanh(gi[:, 2 * FP:] + r * gh[:, 2 * FP:])
    af_nat = afT_all.T
    hnew = (1.0 - z) * n + z * af_nat
    out_ref[...] = jnp.maximum(hnew, 0.0).reshape(M, L, FP)


def kernel(atom_list, bond_list, atom_degree_list, bond_degree_list, atom_mask,
           W_atom, b_atom, W_nb, b_nb, W_align, b_align, W_att, b_att,
           W_ih, W_hh, b_ih, b_hh):
    del atom_mask
    ones = jnp.ones((B, 1, L), jnp.float32)
    bf = jnp.bfloat16
    atomsT2 = jnp.concatenate([atom_list.transpose(0, 2, 1), ones], axis=1).astype(bf)
    bondsT2 = jnp.concatenate([bond_list.transpose(0, 2, 1), ones], axis=1).astype(bf)
    aidxT = atom_degree_list.astype(jnp.int32).transpose(0, 2, 1).astype(bf)
    bidxT = bond_degree_list.astype(jnp.int32).transpose(0, 2, 1).astype(bf)

    w_atom2 = jnp.concatenate([W_atom, b_atom[None, :]], axis=0).astype(bf)
    half_bnb = 0.5 * b_nb[None, :]
    wnb_a2 = jnp.concatenate([W_nb[:F_ATOM], half_bnb], axis=0).astype(bf)
    wnb_b2 = jnp.concatenate([W_nb[F_ATOM:], half_bnb], axis=0).astype(bf)
    w1 = W_align[:FP].astype(bf)
    w2 = W_align[FP:].astype(bf)
    b_align2 = b_align.reshape(1, 1)
    b_att_col = b_att.reshape(FP, 1)
    w_att16 = W_att.astype(bf)
    w_ihT = W_ih.T.astype(bf)
    w_hhT = W_hh.T.astype(bf)

    rep = lambda arr: pl.BlockSpec(arr.shape, lambda i: (0,) * arr.ndim)
    row = lambda v: v.reshape(1, -1)

    out = pl.pallas_call(
        _fused_kernel,
        grid=(B // M,),
        in_specs=[
            pl.BlockSpec((M, F_ATOM + 1, L), lambda i: (i, 0, 0)),
            pl.BlockSpec((M, F_BOND + 1, L), lambda i: (i, 0, 0)),
            pl.BlockSpec((M, D, L), lambda i: (i, 0, 0)),
            pl.BlockSpec((M, D, L), lambda i: (i, 0, 0)),
            rep(w_atom2), rep(wnb_a2), rep(wnb_b2),
            rep(w1), rep(w2), rep(b_align2),
            rep(w_att16), rep(b_att_col),
            rep(w_ihT), rep(row(b_ih)),
            rep(w_hhT), rep(row(b_hh)),
        ],
        out_specs=pl.BlockSpec((M, L, FP), lambda i: (i, 0, 0)),
        out_shape=jax.ShapeDtypeStruct((B, L, FP), jnp.float32),
    )(atomsT2, bondsT2, aidxT, bidxT,
      w_atom2, wnb_a2, wnb_b2, w1, w2, b_align2, w_att16, b_att_col,
      w_ihT, row(b_ih), w_hhT, row(b_hh))
    return out
